# Initial kernel scaffold; baseline (speedup 1.0000x reference)
import jax
import jax.numpy as jnp
from jax.experimental import pallas as pl


def kernel(x_seq, edge_index_seq, W1_rel, b1, W1_root, W2_rel, b2, W2_root, W_ih, W_hh, b_ih, b_hh, W_fc, b_fc):
    raise NotImplementedError("write your pallas kernel here")


# speedup vs baseline: 40.7807x; 40.7807x over previous
import jax
import jax.numpy as jnp
from jax import lax
from jax.experimental import pallas as pl
from jax.experimental.pallas import tpu as pltpu
from jax.experimental.pallas import tpu_sc as plsc

T, N, E = 8, 10000, 320000
NPAD = 10240
QUART = NPAD // 4
TPF = 4
FPC = 4
EPT = E // TPF
CHUNK = 2000
NCH = EPT // CHUNK
RED = QUART // 2


def _sc_segment_sums(xT_pad, edges):
    mesh = plsc.VectorSubcoreMesh(core_axis_name="c", subcore_axis_name="s")

    def body(xt_hbm, e_hbm, out_hbm, xv, acc, ebuf, tmp4, outb, shacc,
             sem0, sem1):
        c = lax.axis_index("c")
        s = lax.axis_index("s")
        t = c * FPC + s // TPF
        p = s % TPF
        g = s // TPF
        slot_base = TPF * g
        sems = (sem0, sem1)

        def zero_acc(nwords):
            def zb(i, _):
                acc[pl.ds(i * 16, 16)] = jnp.zeros((16,), jnp.float32)
                return 0
            lax.fori_loop(0, nwords // 16, zb, 0)

        def fire(k):
            b = k % 2
            h0 = pltpu.async_copy(
                e_hbm.at[pl.ds((t * 2 + 0) * E + p * EPT + k * CHUNK, CHUNK)],
                ebuf.at[b, 0], sems[b])
            h1 = pltpu.async_copy(
                e_hbm.at[pl.ds((t * 2 + 1) * E + p * EPT + k * CHUNK, CHUNK)],
                ebuf.at[b, 1], sems[b])
            return (h0, h1)

        def edge_pass(with_deg):
            ones = jnp.full((16,), 1.0, jnp.float32)
            handles = {0: fire(0), 1: None}
            for k in range(NCH):
                b = k % 2
                if k + 1 < NCH:
                    handles[(k + 1) % 2] = fire(k + 1)
                h0, h1 = handles[b]
                h0.wait()
                h1.wait()

                def ib(i, _):
                    off = i * 16
                    sv = ebuf[b, 0, pl.ds(off, 16)]
                    dv = ebuf[b, 1, pl.ds(off, 16)]
                    for cc in range(3):
                        val = plsc.load_gather(xv, [sv + cc * NPAD])
                        plsc.addupdate_scatter(acc, [dv + cc * NPAD], val)
                    if with_deg:
                        plsc.addupdate_scatter(acc, [dv + 3 * NPAD], ones)
                    return 0
                lax.fori_loop(0, CHUNK // 16, ib, 0)

        def reduce_feature(src_col, dst_row):
            for r in range(QUART // RED):
                for q in range(TPF):
                    pltpu.sync_copy(
                        shacc.at[pl.ds((slot_base + q) * 4 * NPAD
                                       + src_col * NPAD + p * QUART + r * RED,
                                       RED)],
                        tmp4.at[pl.ds(q * RED, RED)])

                def rb(i, _):
                    off = i * 16
                    v = (tmp4[pl.ds(off, 16)] + tmp4[pl.ds(RED + off, 16)]
                         + tmp4[pl.ds(2 * RED + off, 16)]
                         + tmp4[pl.ds(3 * RED + off, 16)])
                    outb[pl.ds(off, 16)] = v
                    return 0
                lax.fori_loop(0, RED // 16, rb, 0)
                pltpu.sync_copy(
                    outb,
                    out_hbm.at[pl.ds((t * 7 + dst_row) * NPAD + p * QUART
                                     + r * RED, RED)])

        for cc in range(3):
            pltpu.sync_copy(xt_hbm.at[pl.ds((t * 3 + cc) * NPAD, NPAD)],
                            xv.at[pl.ds(cc * NPAD, NPAD)])
        zero_acc(4 * NPAD)
        edge_pass(True)
        pltpu.sync_copy(acc, shacc.at[pl.ds(s * 4 * NPAD, 4 * NPAD)])
        plsc.subcore_barrier()
        for cc in range(4):
            reduce_feature(cc, cc)
        plsc.subcore_barrier()

        for cc in range(3):
            pltpu.sync_copy(out_hbm.at[pl.ds((t * 7 + cc) * NPAD, NPAD)],
                            xv.at[pl.ds(cc * NPAD, NPAD)])
        zero_acc(3 * NPAD)
        edge_pass(False)
        pltpu.sync_copy(acc.at[pl.ds(0, 3 * NPAD)],
                        shacc.at[pl.ds(s * 4 * NPAD, 3 * NPAD)])
        plsc.subcore_barrier()
        for cc in range(3):
            reduce_feature(cc, 4 + cc)

    out_flat = pl.kernel(
        body,
        out_type=jax.ShapeDtypeStruct((T * 7 * NPAD,), jnp.float32),
        mesh=mesh,
        compiler_params=pltpu.CompilerParams(use_tc_tiling_on_sc=False,
                                             needs_layout_passes=False),
        scratch_types=[
            pltpu.VMEM((3 * NPAD,), jnp.float32),
            pltpu.VMEM((4 * NPAD,), jnp.float32),
            pltpu.VMEM((2, 2, CHUNK), jnp.int32),
            pltpu.VMEM((4 * RED,), jnp.float32),
            pltpu.VMEM((RED,), jnp.float32),
            pltpu.VMEM_SHARED((16 * 4 * NPAD,), jnp.float32),
            pltpu.SemaphoreType.DMA,
            pltpu.SemaphoreType.DMA,
        ],
    )(xT_pad.reshape(-1), edges.reshape(-1))
    return out_flat.reshape(T, 7, NPAD)


def _tc_finish(xT_pad, sc_out, W1_rel, b1, W1_root, W2_rel, b2, W2_root,
               W_ih, W_hh, b_ih, b_hh, W_fc, b_fc):
    def body(xT_ref, sc_ref, W1r, b1r, W1o, W2r, b2r, W2o, Wih, Whh, bih, bhh,
             Wfc, bfc, out_ref):
        W1_rel_ = W1r[...]
        W1_root_ = W1o[...]
        W2_rel_ = W2r[...]
        W2_root_ = W2o[...]
        b1c = b1r[...][:, None]
        b2c = b2r[...][:, None]
        mask = lax.broadcasted_iota(jnp.int32, (1, NPAD), 1) < N
        embs = []
        for t in range(T):
            xt = xT_ref[t]
            s1 = sc_ref[t, 0:3]
            dg = sc_ref[t, 3:4]
            s2 = sc_ref[t, 4:7]
            h1 = jnp.dot(W1_rel_, s1) + b1c + jnp.dot(W1_root_, xt)
            ah1 = jnp.dot(W1_rel_, s2) + b1c * dg + jnp.dot(W1_root_, s1)
            h2 = jnp.dot(W2_rel_, ah1) + b2c + jnp.dot(W2_root_, h1)
            h2 = jnp.where(mask, jnp.maximum(h2, 0.0), 0.0)
            embs.append(jnp.sum(h2, axis=1)[None, :] * (1.0 / N))
        Wih_ = Wih[...]
        Whh_ = Whh[...]
        bb = bih[...][None, :] + bhh[...][None, :]
        hs = jnp.zeros((1, 128), jnp.float32)
        cs = jnp.zeros((1, 128), jnp.float32)
        dn = (((1,), (1,)), ((), ()))
        for t in range(T):
            gates = (lax.dot_general(embs[t], Wih_, dn)
                     + lax.dot_general(hs, Whh_, dn) + bb)
            ig = jax.nn.sigmoid(gates[:, 0:128])
            fg = jax.nn.sigmoid(gates[:, 128:256])
            gg = jnp.tanh(gates[:, 256:384])
            og = jax.nn.sigmoid(gates[:, 384:512])
            cs = fg * cs + ig * gg
            hs = og * jnp.tanh(cs)
        out_ref[...] = (lax.dot_general(hs, Wfc[...], dn)
                        + bfc[...][None, :])

    return pl.pallas_call(
        body,
        out_shape=jax.ShapeDtypeStruct((1, 5), jnp.float32),
    )(xT_pad, sc_out, W1_rel, b1, W1_root, W2_rel, b2, W2_root,
      W_ih, W_hh, b_ih, b_hh, W_fc, b_fc)


def kernel(x_seq, edge_index_seq, W1_rel, b1, W1_root, W2_rel, b2, W2_root,
           W_ih, W_hh, b_ih, b_hh, W_fc, b_fc):
    xT = jnp.transpose(x_seq, (0, 2, 1))
    xT_pad = jnp.pad(xT, ((0, 0), (0, 0), (0, NPAD - N)))
    edges = edge_index_seq.astype(jnp.int32)
    sc_out = _sc_segment_sums(xT_pad, edges)
    return _tc_finish(xT_pad, sc_out, W1_rel, b1, W1_root, W2_rel, b2,
                      W2_root, W_ih, W_hh, b_ih, b_hh, W_fc, b_fc)


# speedup vs baseline: 44.4326x; 1.0895x over previous
import jax
import jax.numpy as jnp
from jax import lax
from jax.experimental import pallas as pl
from jax.experimental.pallas import tpu as pltpu
from jax.experimental.pallas import tpu_sc as plsc

T, N, E = 8, 10000, 320000
NPAD = 10240
QUART = NPAD // 4
TPF = 4
FPC = 4
EPT = E // TPF
CHUNK = 1600
NCH = EPT // CHUNK
ROWS = 19
S1OFF = 16 * 3 * NPAD


def _sc_segment_sums(xT_pad, edges):
    mesh = plsc.VectorSubcoreMesh(core_axis_name="c", subcore_axis_name="s")

    def body(xt_hbm, e_hbm, out_hbm, xv0, xv1, xv2, acc0, acc1, acc2, accd,
             ebuf, tmp4, outb, shacc, sem0, sem1):
        c = lax.axis_index("c")
        s = lax.axis_index("s")
        t = c * FPC + s // TPF
        p = s % TPF
        g = s // TPF
        slot_base = TPF * g
        sems = (sem0, sem1)
        xvs = (xv0, xv1, xv2)
        accs = (acc0, acc1, acc2)

        def zero_accs(with_deg):
            bufs = accs + ((accd,) if with_deg else ())
            def zb(i, _):
                off = i * 64
                for bf in bufs:
                    for u in range(4):
                        bf[pl.ds(off + u * 16, 16)] = jnp.zeros((16,),
                                                                jnp.float32)
                return 0
            lax.fori_loop(0, NPAD // 64, zb, 0)

        def fire(k):
            b = k % 2
            h0 = pltpu.async_copy(
                e_hbm.at[pl.ds((t * 2 + 0) * E + p * EPT + k * CHUNK, CHUNK)],
                ebuf.at[b, 0], sems[b])
            h1 = pltpu.async_copy(
                e_hbm.at[pl.ds((t * 2 + 1) * E + p * EPT + k * CHUNK, CHUNK)],
                ebuf.at[b, 1], sems[b])
            return (h0, h1)

        def edge_pass(with_deg):
            ones = jnp.full((16,), 1.0, jnp.float32)
            handles = {0: fire(0), 1: None}
            for k in range(NCH):
                b = k % 2
                if k + 1 < NCH:
                    handles[(k + 1) % 2] = fire(k + 1)
                h0, h1 = handles[b]
                h0.wait()
                h1.wait()

                def ib(i, _):
                    for u in range(4):
                        off = i * 64 + u * 16
                        sv = ebuf[b, 0, pl.ds(off, 16)]
                        dv = ebuf[b, 1, pl.ds(off, 16)]
                        for cc in range(3):
                            val = plsc.load_gather(xvs[cc], [sv])
                            plsc.addupdate_scatter(accs[cc], [dv], val)
                        if with_deg:
                            plsc.addupdate_scatter(accd, [dv], ones)
                    return 0
                lax.fori_loop(0, CHUNK // 64, ib, 0)

        for cc in range(3):
            pltpu.sync_copy(xt_hbm.at[pl.ds((t * 3 + cc) * NPAD, NPAD)],
                            xvs[cc].at[...])
        zero_accs(True)
        edge_pass(True)

        for cc in range(3):
            pltpu.sync_copy(accs[cc].at[...],
                            shacc.at[pl.ds((s * 3 + cc) * NPAD, NPAD)])
        pltpu.sync_copy(accd.at[...],
                        out_hbm.at[pl.ds((t * ROWS + 3 + p) * NPAD, NPAD)])
        plsc.subcore_barrier()

        for cc in range(3):
            for q in range(TPF):
                pltpu.sync_copy(
                    shacc.at[pl.ds(((slot_base + q) * 3 + cc) * NPAD
                                   + p * QUART, QUART)],
                    tmp4.at[pl.ds(q * QUART, QUART)])

            def rb(i, _):
                off = i * 16
                v = (tmp4[pl.ds(off, 16)] + tmp4[pl.ds(QUART + off, 16)]
                     + tmp4[pl.ds(2 * QUART + off, 16)]
                     + tmp4[pl.ds(3 * QUART + off, 16)])
                outb[pl.ds(off, 16)] = v
                return 0
            lax.fori_loop(0, QUART // 16, rb, 0)
            pltpu.sync_copy(
                outb.at[...],
                out_hbm.at[pl.ds((t * ROWS + cc) * NPAD + p * QUART, QUART)])
            pltpu.sync_copy(
                outb.at[...],
                shacc.at[pl.ds(S1OFF + (g * 3 + cc) * NPAD + p * QUART,
                               QUART)])
        plsc.subcore_barrier()

        for cc in range(3):
            pltpu.sync_copy(shacc.at[pl.ds(S1OFF + (g * 3 + cc) * NPAD,
                                           NPAD)],
                            xvs[cc].at[...])
        zero_accs(False)
        edge_pass(False)

        for cc in range(3):
            pltpu.sync_copy(
                accs[cc].at[...],
                out_hbm.at[pl.ds((t * ROWS + 7 + p * 3 + cc) * NPAD, NPAD)])

    out_flat = pl.kernel(
        body,
        out_type=jax.ShapeDtypeStruct((T * ROWS * NPAD,), jnp.float32),
        mesh=mesh,
        compiler_params=pltpu.CompilerParams(use_tc_tiling_on_sc=False,
                                             needs_layout_passes=False),
        scratch_types=[
            pltpu.VMEM((NPAD,), jnp.float32),
            pltpu.VMEM((NPAD,), jnp.float32),
            pltpu.VMEM((NPAD,), jnp.float32),
            pltpu.VMEM((NPAD,), jnp.float32),
            pltpu.VMEM((NPAD,), jnp.float32),
            pltpu.VMEM((NPAD,), jnp.float32),
            pltpu.VMEM((NPAD,), jnp.float32),
            pltpu.VMEM((2, 2, CHUNK), jnp.int32),
            pltpu.VMEM((4 * QUART,), jnp.float32),
            pltpu.VMEM((QUART,), jnp.float32),
            pltpu.VMEM_SHARED((S1OFF + 4 * 3 * NPAD,), jnp.float32),
            pltpu.SemaphoreType.DMA,
            pltpu.SemaphoreType.DMA,
        ],
    )(xT_pad.reshape(-1), edges.reshape(-1))
    return out_flat.reshape(T, ROWS, NPAD)


def _tc_finish(xT_pad, sc_out, W1_rel, b1, W1_root, W2_rel, b2, W2_root,
               W_ih, W_hh, b_ih, b_hh, W_fc, b_fc):
    def body(xT_ref, sc_ref, W1r, b1r, W1o, W2r, b2r, W2o, Wih, Whh, bih, bhh,
             Wfc, bfc, out_ref):
        W1_rel_ = W1r[...]
        W1_root_ = W1o[...]
        W2_rel_ = W2r[...]
        W2_root_ = W2o[...]
        b1c = b1r[...][:, None]
        b2c = b2r[...][:, None]
        mask = lax.broadcasted_iota(jnp.int32, (1, NPAD), 1) < N
        embs = []
        for t in range(T):
            xt = xT_ref[t]
            s1 = sc_ref[t, 0:3]
            dg = (sc_ref[t, 3:4] + sc_ref[t, 4:5]
                  + sc_ref[t, 5:6] + sc_ref[t, 6:7])
            s2 = (sc_ref[t, 7:10] + sc_ref[t, 10:13]
                  + sc_ref[t, 13:16] + sc_ref[t, 16:19])
            h1 = jnp.dot(W1_rel_, s1) + b1c + jnp.dot(W1_root_, xt)
            ah1 = jnp.dot(W1_rel_, s2) + b1c * dg + jnp.dot(W1_root_, s1)
            h2 = jnp.dot(W2_rel_, ah1) + b2c + jnp.dot(W2_root_, h1)
            h2 = jnp.where(mask, jnp.maximum(h2, 0.0), 0.0)
            embs.append(jnp.sum(h2, axis=1)[None, :] * (1.0 / N))
        Wih_ = Wih[...]
        Whh_ = Whh[...]
        bb = bih[...][None, :] + bhh[...][None, :]
        hs = jnp.zeros((1, 128), jnp.float32)
        cs = jnp.zeros((1, 128), jnp.float32)
        dn = (((1,), (1,)), ((), ()))
        for t in range(T):
            gates = (lax.dot_general(embs[t], Wih_, dn)
                     + lax.dot_general(hs, Whh_, dn) + bb)
            ig = jax.nn.sigmoid(gates[:, 0:128])
            fg = jax.nn.sigmoid(gates[:, 128:256])
            gg = jnp.tanh(gates[:, 256:384])
            og = jax.nn.sigmoid(gates[:, 384:512])
            cs = fg * cs + ig * gg
            hs = og * jnp.tanh(cs)
        out_ref[...] = (lax.dot_general(hs, Wfc[...], dn)
                        + bfc[...][None, :])

    return pl.pallas_call(
        body,
        out_shape=jax.ShapeDtypeStruct((1, 5), jnp.float32),
    )(xT_pad, sc_out, W1_rel, b1, W1_root, W2_rel, b2, W2_root,
      W_ih, W_hh, b_ih, b_hh, W_fc, b_fc)


def kernel(x_seq, edge_index_seq, W1_rel, b1, W1_root, W2_rel, b2, W2_root,
           W_ih, W_hh, b_ih, b_hh, W_fc, b_fc):
    xT = jnp.transpose(x_seq, (0, 2, 1))
    xT_pad = jnp.pad(xT, ((0, 0), (0, 0), (0, NPAD - N)))
    edges = edge_index_seq.astype(jnp.int32)
    sc_out = _sc_segment_sums(xT_pad, edges)
    return _tc_finish(xT_pad, sc_out, W1_rel, b1, W1_root, W2_rel, b2,
                      W2_root, W_ih, W_hh, b_ih, b_hh, W_fc, b_fc)


# speedup vs baseline: 45.6487x; 1.0274x over previous
import jax
import jax.numpy as jnp
from jax import lax
from jax.experimental import pallas as pl
from jax.experimental.pallas import tpu as pltpu
from jax.experimental.pallas import tpu_sc as plsc

T, N, E = 8, 10000, 320000
NPAD = 10240
QUART = NPAD // 4
TPF = 4
FPC = 4
EPT = E // TPF
CHUNK = 1600
NCH = EPT // CHUNK
ROWS = 19
S1OFF = 16 * 3 * NPAD


def _sc_segment_sums(xT_pad, edges):
    mesh = plsc.VectorSubcoreMesh(core_axis_name="c", subcore_axis_name="s")

    def body(xt_hbm, e_hbm, out_hbm, xv0, xv1, xv2, acc0, acc1, acc2, accd,
             ebuf, tmp4, outb, shacc, sem0, sem1):
        c = lax.axis_index("c")
        s = lax.axis_index("s")
        t = c * FPC + s // TPF
        p = s % TPF
        g = s // TPF
        slot_base = TPF * g
        sems = (sem0, sem1)
        xvs = (xv0, xv1, xv2)
        accs = (acc0, acc1, acc2)

        def zero_accs(with_deg):
            bufs = accs + ((accd,) if with_deg else ())
            def zb(i, _):
                off = i * 64
                for bf in bufs:
                    for u in range(4):
                        bf[pl.ds(off + u * 16, 16)] = jnp.zeros((16,),
                                                                jnp.float32)
                return 0
            lax.fori_loop(0, NPAD // 64, zb, 0)

        def fire(k):
            b = k % 2
            h0 = pltpu.async_copy(
                e_hbm.at[pl.ds((t * 2 + 0) * E + p * EPT + k * CHUNK, CHUNK)],
                ebuf.at[b, 0], sems[b])
            h1 = pltpu.async_copy(
                e_hbm.at[pl.ds((t * 2 + 1) * E + p * EPT + k * CHUNK, CHUNK)],
                ebuf.at[b, 1], sems[b])
            return (h0, h1)

        def edge_pass(with_deg):
            ones = jnp.full((16,), 1.0, jnp.float32)
            handles = {0: fire(0), 1: None}
            for k in range(NCH):
                b = k % 2
                if k + 1 < NCH:
                    handles[(k + 1) % 2] = fire(k + 1)
                h0, h1 = handles[b]
                h0.wait()
                h1.wait()

                def ib(i, _):
                    for u in range(4):
                        off = i * 64 + u * 16
                        sv = ebuf[b, 0, pl.ds(off, 16)]
                        dv = ebuf[b, 1, pl.ds(off, 16)]
                        for cc in range(3):
                            val = plsc.load_gather(xvs[cc], [sv])
                            plsc.addupdate_scatter(accs[cc], [dv], val)
                        if with_deg:
                            plsc.addupdate_scatter(accd, [dv], ones)
                    return 0
                lax.fori_loop(0, CHUNK // 64, ib, 0)

        for cc in range(3):
            pltpu.sync_copy(xt_hbm.at[pl.ds((t * 3 + cc) * NPAD, NPAD)],
                            xvs[cc].at[...])
        zero_accs(True)
        edge_pass(True)

        for cc in range(3):
            pltpu.sync_copy(accs[cc].at[...],
                            shacc.at[pl.ds((s * 3 + cc) * NPAD, NPAD)])
        pltpu.sync_copy(accd.at[...],
                        out_hbm.at[pl.ds((t * ROWS + 3 + p) * NPAD, NPAD)])
        plsc.subcore_barrier()

        for cc in range(3):
            for q in range(TPF):
                pltpu.sync_copy(
                    shacc.at[pl.ds(((slot_base + q) * 3 + cc) * NPAD
                                   + p * QUART, QUART)],
                    tmp4.at[pl.ds(q * QUART, QUART)])

            def rb(i, _):
                off = i * 16
                v = (tmp4[pl.ds(off, 16)] + tmp4[pl.ds(QUART + off, 16)]
                     + tmp4[pl.ds(2 * QUART + off, 16)]
                     + tmp4[pl.ds(3 * QUART + off, 16)])
                outb[pl.ds(off, 16)] = v
                return 0
            lax.fori_loop(0, QUART // 16, rb, 0)
            pltpu.sync_copy(
                outb.at[...],
                out_hbm.at[pl.ds((t * ROWS + cc) * NPAD + p * QUART, QUART)])
            pltpu.sync_copy(
                outb.at[...],
                shacc.at[pl.ds(S1OFF + (g * 3 + cc) * NPAD + p * QUART,
                               QUART)])
        plsc.subcore_barrier()

        for cc in range(3):
            pltpu.sync_copy(shacc.at[pl.ds(S1OFF + (g * 3 + cc) * NPAD,
                                           NPAD)],
                            xvs[cc].at[...])
        zero_accs(False)
        edge_pass(False)

        for cc in range(3):
            pltpu.sync_copy(
                accs[cc].at[...],
                out_hbm.at[pl.ds((t * ROWS + 7 + p * 3 + cc) * NPAD, NPAD)])

    out_flat = pl.kernel(
        body,
        out_type=jax.ShapeDtypeStruct((T * ROWS * NPAD,), jnp.float32),
        mesh=mesh,
        compiler_params=pltpu.CompilerParams(use_tc_tiling_on_sc=False,
                                             needs_layout_passes=False),
        scratch_types=[
            pltpu.VMEM((NPAD,), jnp.float32),
            pltpu.VMEM((NPAD,), jnp.float32),
            pltpu.VMEM((NPAD,), jnp.float32),
            pltpu.VMEM((NPAD,), jnp.float32),
            pltpu.VMEM((NPAD,), jnp.float32),
            pltpu.VMEM((NPAD,), jnp.float32),
            pltpu.VMEM((NPAD,), jnp.float32),
            pltpu.VMEM((2, 2, CHUNK), jnp.int32),
            pltpu.VMEM((4 * QUART,), jnp.float32),
            pltpu.VMEM((QUART,), jnp.float32),
            pltpu.VMEM_SHARED((S1OFF + 4 * 3 * NPAD,), jnp.float32),
            pltpu.SemaphoreType.DMA,
            pltpu.SemaphoreType.DMA,
        ],
    )(xT_pad.reshape(-1), edges.reshape(-1))
    return out_flat.reshape(T, ROWS, NPAD)


def _tc_finish(xT_pad, sc_out, W1_rel, b1, W1_root, W2_rel, b2, W2_root,
               W_ih, W_hh, b_ih, b_hh, W_fc, b_fc):
    def body(xT_ref, sc_ref, W1r, b1r, W1o, W2r, b2r, W2o, Wih, Whh, bih, bhh,
             Wfc, bfc, out_ref):
        W1_rel_ = W1r[...]
        W1_root_ = W1o[...]
        W2_rel_ = W2r[...]
        W2_root_ = W2o[...]
        b1_ = b1r[...]
        M1 = jnp.dot(W2_rel_, W1_rel_)
        M2 = jnp.dot(W2_rel_, W1_root_) + jnp.dot(W2_root_, W1_rel_)
        M3 = jnp.dot(W2_root_, W1_root_)
        v1 = jnp.dot(W2_rel_, b1_)[:, None]
        cb = b2r[...][:, None] + jnp.dot(W2_root_, b1_)[:, None]
        mask = lax.broadcasted_iota(jnp.int32, (1, NPAD), 1) < N
        embs = []
        for t in range(T):
            xt = xT_ref[t]
            s1 = sc_ref[t, 0:3]
            dg = (sc_ref[t, 3:4] + sc_ref[t, 4:5]
                  + sc_ref[t, 5:6] + sc_ref[t, 6:7])
            s2 = (sc_ref[t, 7:10] + sc_ref[t, 10:13]
                  + sc_ref[t, 13:16] + sc_ref[t, 16:19])
            h2 = (jnp.dot(M3, xt) + jnp.dot(M2, s1) + jnp.dot(M1, s2)
                  + v1 * dg + cb)
            h2 = jnp.where(mask, jnp.maximum(h2, 0.0), 0.0)
            embs.append(jnp.sum(h2, axis=1)[None, :] * (1.0 / N))
        Wih_ = Wih[...]
        Whh_ = Whh[...]
        bb = bih[...][None, :] + bhh[...][None, :]
        hs = jnp.zeros((1, 128), jnp.float32)
        cs = jnp.zeros((1, 128), jnp.float32)
        dn = (((1,), (1,)), ((), ()))
        for t in range(T):
            gates = (lax.dot_general(embs[t], Wih_, dn)
                     + lax.dot_general(hs, Whh_, dn) + bb)
            ig = jax.nn.sigmoid(gates[:, 0:128])
            fg = jax.nn.sigmoid(gates[:, 128:256])
            gg = jnp.tanh(gates[:, 256:384])
            og = jax.nn.sigmoid(gates[:, 384:512])
            cs = fg * cs + ig * gg
            hs = og * jnp.tanh(cs)
        out_ref[...] = (lax.dot_general(hs, Wfc[...], dn)
                        + bfc[...][None, :])

    return pl.pallas_call(
        body,
        out_shape=jax.ShapeDtypeStruct((1, 5), jnp.float32),
    )(xT_pad, sc_out, W1_rel, b1, W1_root, W2_rel, b2, W2_root,
      W_ih, W_hh, b_ih, b_hh, W_fc, b_fc)


def kernel(x_seq, edge_index_seq, W1_rel, b1, W1_root, W2_rel, b2, W2_root,
           W_ih, W_hh, b_ih, b_hh, W_fc, b_fc):
    xT = jnp.transpose(x_seq, (0, 2, 1))
    xT_pad = jnp.pad(xT, ((0, 0), (0, 0), (0, NPAD - N)))
    edges = edge_index_seq.astype(jnp.int32)
    sc_out = _sc_segment_sums(xT_pad, edges)
    return _tc_finish(xT_pad, sc_out, W1_rel, b1, W1_root, W2_rel, b2,
                      W2_root, W_ih, W_hh, b_ih, b_hh, W_fc, b_fc)
